# baseline (speedup 1.0000x reference)
import functools

import numpy as np

import jax
import jax.numpy as jnp
from jax import lax
from jax.experimental import pallas as pl
from jax.experimental.pallas import tpu as pltpu
from jax.experimental.pallas import tpu_sc as plsc

B = 2
INC = 3
H = 32
W_ = 64
C = 256
E = 8
FF = 1024
OUTC = 20
N = B * H * W_
TILE = 128
NPAD = N + E * TILE
NT = NPAD // TILE
DEPTHS = [2, 2]
HEADS = [4, 8]
WINS = [(32, 64), (4, 8)]

_F32 = jnp.float32


def _ln(x, g, b, eps=1e-6):
    m = jnp.mean(x, axis=-1, keepdims=True)
    v = jnp.mean((x - m) ** 2, axis=-1, keepdims=True)
    return (x - m) / jnp.sqrt(v + eps) * g + b


def _embed_body(x_ref, w_ref, b_ref, pos_ref, o_ref):
    o_ref[...] = (
        jnp.dot(x_ref[...], w_ref[...], preferred_element_type=_F32)
        + b_ref[...] + pos_ref[...]
    )


def _embed(xin, pe_w, pe_b, pos):
    rows = 512
    return pl.pallas_call(
        _embed_body,
        grid=(N // rows,),
        in_specs=[
            pl.BlockSpec((rows, 8), lambda i: (i, 0)),
            pl.BlockSpec((8, C), lambda i: (0, 0)),
            pl.BlockSpec((1, C), lambda i: (0, 0)),
            pl.BlockSpec((rows, C), lambda i: (i, 0)),
        ],
        out_specs=pl.BlockSpec((rows, C), lambda i: (i, 0)),
        out_shape=jax.ShapeDtypeStruct((N, C), _F32),
    )(xin, pe_w, pe_b, pos)


def _lnqkv_body_res(x_ref, e_ref, g_ref, lg_ref, lb_ref, w_ref, b_ref,
                    xo_ref, qkv_ref):
    x = x_ref[...] + g_ref[...] * e_ref[...]
    xo_ref[...] = x
    y = _ln(x, lg_ref[...], lb_ref[...])
    qkv_ref[...] = (
        jnp.dot(y, w_ref[...], preferred_element_type=_F32) + b_ref[...]
    )


def _lnqkv_body(x_ref, lg_ref, lb_ref, w_ref, b_ref, xo_ref, qkv_ref):
    x = x_ref[...]
    xo_ref[...] = x
    y = _ln(x, lg_ref[...], lb_ref[...])
    qkv_ref[...] = (
        jnp.dot(y, w_ref[...], preferred_element_type=_F32) + b_ref[...]
    )


def _lnqkv(xin, extra, gate, lg, lb, qkv_w, qkv_b):
    rows = 512
    row_spec = pl.BlockSpec((rows, C), lambda i: (i, 0))
    wspec = [
        pl.BlockSpec((1, C), lambda i: (0, 0)),
        pl.BlockSpec((1, C), lambda i: (0, 0)),
        pl.BlockSpec((C, 3 * C), lambda i: (0, 0)),
        pl.BlockSpec((1, 3 * C), lambda i: (0, 0)),
    ]
    out_specs = [row_spec, pl.BlockSpec((rows, 3 * C), lambda i: (i, 0))]
    out_shape = [jax.ShapeDtypeStruct((N, C), _F32),
                 jax.ShapeDtypeStruct((N, 3 * C), _F32)]
    if extra is None:
        return pl.pallas_call(
            _lnqkv_body, grid=(N // rows,),
            in_specs=[row_spec] + wspec,
            out_specs=out_specs, out_shape=out_shape,
        )(xin, lg.reshape(1, C), lb.reshape(1, C), qkv_w,
          qkv_b.reshape(1, 3 * C))
    return pl.pallas_call(
        _lnqkv_body_res, grid=(N // rows,),
        in_specs=[row_spec, row_spec,
                  pl.BlockSpec((rows, 1), lambda i: (i, 0))] + wspec,
        out_specs=out_specs, out_shape=out_shape,
    )(xin, extra, gate, lg.reshape(1, C), lb.reshape(1, C), qkv_w,
      qkv_b.reshape(1, 3 * C))


def _attn_big_e_body(q_ref, k_ref, e_ref, *, scale):
    q = q_ref[0, 0]
    k = k_ref[0, 0]
    s = lax.dot_general(q, k, (((1,), (1,)), ((), ())),
                        preferred_element_type=_F32) / scale
    m = jnp.max(s, axis=-1, keepdims=True)
    e_ref[0, 0] = jnp.exp(s - m)


def _attn_big_o_body(e_ref, d_ref, v_ref, o_ref):
    p = e_ref[0, 0] / d_ref[0, 0]
    v = v_ref[0, 0]
    kk = v.shape[0]
    def dot(i):
        return lax.dot_general(v[i * 128:(i + 1) * 128],
                               p[:, i * 128:(i + 1) * 128],
                               (((0,), (1,)), ((), ())),
                               preferred_element_type=_F32)
    a0 = dot(0)
    a1 = dot(1)
    for i in range(2, kk // 128):
        if i % 2 == 0:
            a0 = a0 + dot(i)
        else:
            a1 = a1 + dot(i)
    o_ref[0, 0] = a0 + a1


def _attn_big(q, k, v, nb, heads, l, dh):
    rows = 512
    ebody = functools.partial(_attn_big_e_body,
                              scale=float(np.sqrt(np.float32(dh))))
    e = pl.pallas_call(
        ebody,
        grid=(nb, heads, l // rows),
        in_specs=[
            pl.BlockSpec((1, 1, rows, dh), lambda w, h, i: (w, h, i, 0)),
            pl.BlockSpec((1, 1, l, dh), lambda w, h, i: (w, h, 0, 0)),
        ],
        out_specs=pl.BlockSpec((1, 1, rows, l), lambda w, h, i: (w, h, i, 0)),
        out_shape=jax.ShapeDtypeStruct((nb, heads, l, l), _F32),
    )(q, k)
    den = jnp.sum(e, axis=-1, keepdims=True)
    ot = pl.pallas_call(
        _attn_big_o_body,
        grid=(nb, heads, l // rows),
        in_specs=[
            pl.BlockSpec((1, 1, rows, l), lambda w, h, i: (w, h, i, 0)),
            pl.BlockSpec((1, 1, rows, 1), lambda w, h, i: (w, h, i, 0)),
            pl.BlockSpec((1, 1, l, dh), lambda w, h, i: (w, h, 0, 0)),
        ],
        out_specs=pl.BlockSpec((1, 1, dh, rows), lambda w, h, i: (w, h, 0, i)),
        out_shape=jax.ShapeDtypeStruct((nb, heads, dh, l), _F32),
    )(e, den, v)
    return ot.transpose(0, 1, 3, 2)


def _attn_small_e_body(q_ref, k_ref, e_ref, *, scale, l, wb):
    hl = q_ref.shape[1]
    heads = hl // l
    rh = jax.lax.broadcasted_iota(jnp.int32, (hl, hl), 0) // l
    ch = jax.lax.broadcasted_iota(jnp.int32, (hl, hl), 1) // l
    same = rh == ch
    for w in range(wb):
        q = q_ref[w]
        k = k_ref[w]
        s = lax.dot_general(q, k, (((1,), (1,)), ((), ())),
                            preferred_element_type=_F32) / scale
        s = jnp.where(same, s, -1e30)
        m = jnp.max(s, axis=-1, keepdims=True)
        e = jnp.exp(s - m)
        e_ref[w] = jnp.concatenate(
            [e[h * l:(h + 1) * l, h * l:(h + 1) * l] for h in range(heads)],
            axis=0)


def _attn_small_o_body(e_ref, d_ref, v_ref, o_ref, *, l, wb):
    hl = e_ref.shape[1]
    heads = hl // l
    rh = jax.lax.broadcasted_iota(jnp.int32, (hl, hl), 0) // l
    ch = jax.lax.broadcasted_iota(jnp.int32, (hl, hl), 1) // l
    same = rh == ch
    for w in range(wb):
        p = e_ref[w] / d_ref[w]
        pt = jnp.concatenate([p] * heads, axis=1)
        pfull = jnp.where(same, pt, 0.0)
        o_ref[w] = lax.dot_general(pfull, v_ref[w], (((1,), (0,)), ((), ())),
                                   preferred_element_type=_F32)


def _attn_small(q, k, v, nb, heads, l, dh):
    wb = 16
    hl = heads * l
    ebody = functools.partial(_attn_small_e_body,
                              scale=float(np.sqrt(np.float32(dh))), l=l, wb=wb)
    spec_qk = pl.BlockSpec((wb, hl, dh), lambda i: (i, 0, 0))
    e = pl.pallas_call(
        ebody,
        grid=(nb // wb,),
        in_specs=[spec_qk, spec_qk],
        out_specs=pl.BlockSpec((wb, hl, l), lambda i: (i, 0, 0)),
        out_shape=jax.ShapeDtypeStruct((nb, hl, l), _F32),
    )(q, k)
    den = jnp.sum(e.reshape(nb, heads, l, l), axis=-1, keepdims=True)
    den = den.reshape(nb, hl, 1)
    obody = functools.partial(_attn_small_o_body, l=l, wb=wb)
    return pl.pallas_call(
        obody,
        grid=(nb // wb,),
        in_specs=[
            pl.BlockSpec((wb, hl, l), lambda i: (i, 0, 0)),
            pl.BlockSpec((wb, hl, 1), lambda i: (i, 0, 0)),
            spec_qk,
        ],
        out_specs=spec_qk,
        out_shape=jax.ShapeDtypeStruct((nb, hl, dh), _F32),
    )(e, den, v)


def _router_math(o, attr, rw_o, rw_a, rb,
                 dst_ref, gate_ref, te_ref, z_ref, bal_ref, oh_ref):
    ab = attr.astype(jnp.bfloat16).astype(_F32)
    wb = rw_a.astype(jnp.bfloat16).astype(_F32)
    logits = (jnp.dot(o, rw_o, preferred_element_type=_F32)
              + (ab[:, 0:1] * wb[0:1, :] + ab[:, 1:2] * wb[1:2, :])) + rb
    maxv = jnp.max(logits, axis=-1, keepdims=True)
    eiota = jax.lax.broadcasted_iota(jnp.int32, (N, E), 1)
    first = jnp.min(jnp.where(logits == maxv, eiota, E), axis=-1,
                    keepdims=True)
    onehot = (eiota == first).astype(_F32)
    el = jnp.exp(logits - maxv)
    denom = jnp.sum(el, axis=-1, keepdims=True)
    probs = el / denom
    gate_ref[...] = jnp.sum(probs * onehot, axis=-1, keepdims=True)
    lse = maxv + jnp.log(denom)
    z_ref[...] = jnp.sum(lse * lse, axis=0, keepdims=True) / N
    counts = jnp.sum(onehot, axis=0, keepdims=True)
    psum = jnp.sum(probs, axis=0, keepdims=True)
    bal_ref[...] = ((E / (N * float(N)))
                    * jnp.sum(counts * psum, axis=1, keepdims=True))
    pc = jnp.ceil(counts / TILE) * TILE
    starts_l = [jnp.zeros((1, 1), _F32)]
    for e in range(E - 1):
        starts_l.append(starts_l[-1] + pc[:, e:e + 1])
    starts = jnp.concatenate(starts_l, axis=1)
    ti = jax.lax.broadcasted_iota(jnp.int32, (1, NT), 1).astype(_F32) * TILE
    te = jnp.zeros((1, NT), _F32)
    for e in range(E):
        te += (ti >= starts[:, e:e + 1]).astype(_F32)
    te_ref[...] = te - 1.0
    oh_ref[...] = onehot
    tri = (jax.lax.broadcasted_iota(jnp.int32, (TILE, TILE), 0)
           >= jax.lax.broadcasted_iota(jnp.int32, (TILE, TILE), 1)
           ).astype(_F32)

    def chunk(i, run):
        oh = oh_ref[pl.ds(i * TILE, TILE), :]
        cs = jnp.dot(tri, oh, preferred_element_type=_F32) + run - 1.0
        slot = jnp.sum((starts + cs) * oh, axis=-1, keepdims=True)
        dst_ref[pl.ds(i * TILE, TILE), :] = slot.astype(jnp.int32)
        return run + jnp.sum(oh, axis=0, keepdims=True)

    lax.fori_loop(0, N // TILE, chunk, jnp.zeros((1, E), _F32))


def _router_plain_body(o_ref, attr_ref, rwo_ref, rwa_ref, rb_ref,
                       dst_ref, gate_ref, te_ref, z_ref, bal_ref, oh_ref):
    _router_math(o_ref[...], attr_ref[...], rwo_ref[...], rwa_ref[...],
                 rb_ref[...], dst_ref, gate_ref, te_ref, z_ref, bal_ref,
                 oh_ref)


def _router_fused_body(x_ref, y2_ref, g_ref, lg_ref, lb_ref,
                       attr_ref, rwo_ref, rwa_ref, rb_ref,
                       xo_ref, lnt_ref, dst_ref, gate_ref, te_ref,
                       z_ref, bal_ref, oh_ref):
    x = x_ref[...] + g_ref[...] * y2_ref[...]
    xo_ref[...] = x
    lnt = _ln(x, lg_ref[...], lb_ref[...])
    lnt_ref[...] = lnt
    _router_math(lnt, attr_ref[...], rwo_ref[...], rwa_ref[...],
                 rb_ref[...], dst_ref, gate_ref, te_ref, z_ref, bal_ref,
                 oh_ref)


_ROUTER_OUTS = [
    jax.ShapeDtypeStruct((N, 1), jnp.int32),
    jax.ShapeDtypeStruct((N, 1), _F32),
    jax.ShapeDtypeStruct((1, NT), _F32),
    jax.ShapeDtypeStruct((1, 1), _F32),
    jax.ShapeDtypeStruct((1, 1), _F32),
]


def _router_plain(o, attr, rw, rb):
    return pl.pallas_call(
        _router_plain_body,
        out_shape=list(_ROUTER_OUTS),
        scratch_shapes=[pltpu.VMEM((N, E), _F32)],
    )(o, attr, rw[:C], rw[C:], rb.reshape(1, E))


def _router_fused(x, y2, gate_in, lg, lb, attr, rw, rb):
    return pl.pallas_call(
        _router_fused_body,
        out_shape=[jax.ShapeDtypeStruct((N, C), _F32),
                   jax.ShapeDtypeStruct((N, C), _F32)] + list(_ROUTER_OUTS),
        scratch_shapes=[pltpu.VMEM((N, E), _F32)],
    )(x, y2, gate_in, lg.reshape(1, C), lb.reshape(1, C), attr,
      rw[:C], rw[C:], rb.reshape(1, E))


_NW = 32
_PER = N // _NW


@functools.lru_cache(maxsize=None)
def _sc_kernels():
    info = plsc.get_sparse_core_info()
    nc = info.num_cores
    mesh = plsc.VectorSubcoreMesh(core_axis_name="c", subcore_axis_name="s")
    scratch = [pltpu.VMEM((_PER,), jnp.int32),
               pltpu.VMEM((_PER, C), _F32),
               pltpu.SemaphoreType.DMA]

    @functools.partial(
        pl.kernel, mesh=mesh,
        out_type=jax.ShapeDtypeStruct((NPAD, C), _F32),
        scratch_types=scratch)
    def dispatch(tok_hbm, dst_hbm, out_hbm, idx_v, rows_v, sem):
        wid = lax.axis_index("s") * nc + lax.axis_index("c")
        base = wid * _PER
        pltpu.sync_copy(dst_hbm.at[pl.ds(base, _PER)], idx_v)
        pltpu.sync_copy(tok_hbm.at[pl.ds(base, _PER)], rows_v)
        pltpu.async_copy(rows_v, out_hbm.at[idx_v], sem).wait()

    @functools.partial(
        pl.kernel, mesh=mesh,
        out_type=jax.ShapeDtypeStruct((N, C), _F32),
        scratch_types=scratch)
    def combine(ybuf_hbm, dst_hbm, out_hbm, idx_v, rows_v, sem):
        wid = lax.axis_index("s") * nc + lax.axis_index("c")
        base = wid * _PER
        pltpu.sync_copy(dst_hbm.at[pl.ds(base, _PER)], idx_v)
        pltpu.async_copy(ybuf_hbm.at[idx_v], rows_v, sem).wait()
        pltpu.sync_copy(rows_v, out_hbm.at[pl.ds(base, _PER)])

    return dispatch, combine


def _dispatch(tok, dst):
    return _sc_kernels()[0](tok, dst)


def _combine(ybuf, dst):
    return _sc_kernels()[1](ybuf, dst)


def _gmm_proj_body(te_ref, x_ref, w_ref, b_ref, o_ref):
    del te_ref
    o_ref[...] = (
        jnp.dot(x_ref[...], w_ref[0], preferred_element_type=_F32)
        + b_ref[0]
    )


def _gmm_proj(te, xbuf, w, b):
    return pl.pallas_call(
        _gmm_proj_body,
        grid_spec=pltpu.PrefetchScalarGridSpec(
            num_scalar_prefetch=1,
            grid=(NT,),
            in_specs=[
                pl.BlockSpec((TILE, C), lambda t, te: (t, 0)),
                pl.BlockSpec((1, C, C), lambda t, te: (te[t], 0, 0)),
                pl.BlockSpec((1, 1, C), lambda t, te: (te[t], 0, 0)),
            ],
            out_specs=pl.BlockSpec((TILE, C), lambda t, te: (t, 0)),
        ),
        out_shape=jax.ShapeDtypeStruct((NPAD, C), _F32),
    )(te, xbuf, w, b.reshape(E, 1, C))


def _gmm_mlp_body(te_ref, x_ref, w1_ref, b1_ref, w2_ref, b2_ref, o_ref):
    del te_ref
    h = jax.nn.gelu(
        jnp.dot(x_ref[...], w1_ref[0], preferred_element_type=_F32)
        + b1_ref[0])
    w2 = w2_ref[0]
    hw = (jnp.dot(h[:, :512], w2[:512], preferred_element_type=_F32)
          + jnp.dot(h[:, 512:], w2[512:], preferred_element_type=_F32))
    o_ref[...] = hw + b2_ref[0]


def _gmm_mlp(te, xbuf, w1, b1, w2, b2):
    return pl.pallas_call(
        _gmm_mlp_body,
        grid_spec=pltpu.PrefetchScalarGridSpec(
            num_scalar_prefetch=1,
            grid=(NT,),
            in_specs=[
                pl.BlockSpec((TILE, C), lambda t, te: (t, 0)),
                pl.BlockSpec((1, C, FF), lambda t, te: (te[t], 0, 0)),
                pl.BlockSpec((1, 1, FF), lambda t, te: (te[t], 0, 0)),
                pl.BlockSpec((1, FF, C), lambda t, te: (te[t], 0, 0)),
                pl.BlockSpec((1, 1, C), lambda t, te: (te[t], 0, 0)),
            ],
            out_specs=pl.BlockSpec((TILE, C), lambda t, te: (t, 0)),
        ),
        out_shape=jax.ShapeDtypeStruct((NPAD, C), _F32),
    )(te, xbuf, w1, b1.reshape(E, 1, FF), w2, b2.reshape(E, 1, C))


def _final_body(x_ref, m_ref, g_ref, w_ref, o_ref):
    x = x_ref[...] + g_ref[...] * m_ref[...]
    o_ref[...] = jnp.dot(x, w_ref[...], preferred_element_type=_F32)


def _final(xr, m, gate, fw):
    rows = 512
    return pl.pallas_call(
        _final_body,
        grid=(N // rows,),
        in_specs=[
            pl.BlockSpec((rows, C), lambda i: (i, 0)),
            pl.BlockSpec((rows, C), lambda i: (i, 0)),
            pl.BlockSpec((rows, 1), lambda i: (i, 0)),
            pl.BlockSpec((C, OUTC), lambda i: (0, 0)),
        ],
        out_specs=pl.BlockSpec((rows, OUTC), lambda i: (i, 0)),
        out_shape=jax.ShapeDtypeStruct((N, OUTC), _F32),
    )(xr, m, gate, fw)


def _win_partition(x4, wh, ww):
    b, h, w, c = x4.shape
    x = x4.reshape(b, h // wh, wh, w // ww, ww, c)
    return x.transpose(0, 1, 3, 2, 4, 5).reshape(-1, wh * ww, c)


def _win_reverse(xw, wh, ww, h, w):
    c = xw.shape[-1]
    b = xw.shape[0] // ((h // wh) * (w // ww))
    x = xw.reshape(b, h // wh, w // ww, wh, ww, c)
    return x.transpose(0, 1, 3, 2, 4, 5).reshape(b, h, w, c)


def _make_coord():
    ys = -1.0 + 1.0 / H + (2.0 / H) * jnp.arange(H, dtype=_F32)
    xs = -1.0 + 1.0 / W_ + (2.0 / W_) * jnp.arange(W_, dtype=_F32)
    yy, xx = jnp.meshgrid(ys, xs, indexing='ij')
    return jnp.stack([yy, xx], axis=-1)


def _block(x, extra, gate_prev, attr4, p, heads, win, shift):
    wh, ww = win
    l = wh * ww
    nb = N // l
    dh = C // heads

    xnew, qkv = _lnqkv(x, extra, gate_prev, p['ln1_g'], p['ln1_b'],
                       p['qkv_w'], p['qkv_b'])

    qkv4 = qkv.reshape(B, H, W_, 3 * C)
    a4 = attr4
    if shift[0] or shift[1]:
        qkv4 = jnp.roll(qkv4, (-shift[0], -shift[1]), axis=(1, 2))
        a4 = jnp.roll(a4, (-shift[0], -shift[1]), axis=(1, 2))
    qkvw = _win_partition(qkv4, wh, ww)
    aw = _win_partition(a4, wh, ww).reshape(nb * l, 2)

    qkv5 = qkvw.reshape(nb, l, 3, heads, dh)
    if l == 2048:
        qkv5 = qkv5.transpose(2, 0, 3, 1, 4)
        o = _attn_big(qkv5[0], qkv5[1], qkv5[2], nb, heads, l, dh)
        o = o.transpose(0, 2, 1, 3).reshape(nb * l, C)
    else:
        qkv5 = qkv5.transpose(2, 0, 3, 1, 4).reshape(3, nb, heads * l, dh)
        o = _attn_small(qkv5[0], qkv5[1], qkv5[2], nb, heads, l, dh)
        o = (o.reshape(nb, heads, l, dh).transpose(0, 2, 1, 3)
             .reshape(nb * l, C))

    dst1, gate1, te1f, z1, b1 = _router_plain(o, aw, p['proj_rw'],
                                              p['proj_rb'])
    te1 = te1f.reshape(NT).astype(jnp.int32)
    d1 = dst1.reshape(N)
    xbuf = _dispatch(o, d1)
    ybuf = _gmm_proj(te1, xbuf, p['proj_w'], p['proj_b'])
    yt = _combine(ybuf, d1)

    y2 = _win_reverse(yt.reshape(nb, l, C), wh, ww, H, W_)
    g1 = _win_reverse(gate1.reshape(nb, l, 1), wh, ww, H, W_)
    if shift[0] or shift[1]:
        y2 = jnp.roll(y2, (shift[0], shift[1]), axis=(1, 2))
        g1 = jnp.roll(g1, (shift[0], shift[1]), axis=(1, 2))
    y2 = y2.reshape(N, C)
    g1 = g1.reshape(N, 1)

    attr2 = attr4.reshape(N, 2)
    (xr, lnt, dst2, gate2, te2f, z2, b2) = _router_fused(
        xnew, y2, g1, p['ln2_g'], p['ln2_b'], attr2,
        p['mlp_rw'], p['mlp_rb'])
    te2 = te2f.reshape(NT).astype(jnp.int32)
    d2 = dst2.reshape(N)
    xbuf2 = _dispatch(lnt, d2)
    ybuf2 = _gmm_mlp(te2, xbuf2, p['fc1_w'], p['fc1_b'],
                     p['fc2_w'], p['fc2_b'])
    m = _combine(ybuf2, d2)

    zs = [z1[0, 0], z2[0, 0]]
    bs = [b1[0, 0], b2[0, 0]]
    return xr, m, gate2, zs, bs


def kernel(x, index, params):
    del index
    xin = x.transpose(0, 2, 3, 1).reshape(N, INC)
    xin = jnp.pad(xin, ((0, 0), (0, 8 - INC)))
    pe_w = jnp.pad(params['pe_w'], ((0, 8 - INC), (0, 0)))
    pos = jnp.broadcast_to(params['pos_embed'], (B, H * W_, C)).reshape(N, C)
    xt = _embed(xin, pe_w, params['pe_b'].reshape(1, C), pos)

    coord = jnp.clip(_make_coord(), -1 + 1e-6, 1 - 1e-6)
    attr4 = jnp.broadcast_to(coord[None], (B, H, W_, 2))

    shifts = [[(0, 0), (0, 0)],
              [(0, 0), (WINS[1][0] // 2, WINS[1][1] // 2)]]

    xcur, extra, gate_prev = xt, None, None
    zl, bl = [], []
    for li in range(len(DEPTHS)):
        for bi in range(DEPTHS[li]):
            xcur, extra, gate_prev, zs, bs = _block(
                xcur, extra, gate_prev, attr4,
                params['layers'][li][bi], HEADS[li], WINS[li],
                shifts[li][bi])
            zl += zs
            bl += bs

    res = _final(xcur, extra, gate_prev, params['final_w'])
    res = res.reshape(B, H, W_, OUTC).transpose(0, 3, 1, 2)
    return res, jnp.stack(zl), jnp.stack(bl)
